# Initial kernel scaffold; baseline (speedup 1.0000x reference)
import jax
import jax.numpy as jnp
from jax.experimental import pallas as pl


def kernel(x, edge_index, W1_0, b1_0, W2_0, b2_0, W1_1, b1_1, W2_1, b2_1, W1_2, b1_2, W2_2, b2_2, Wf1, bf1, Wf2, bf2):
    raise NotImplementedError("write your pallas kernel here")


# speedup vs baseline: 7.0934x; 7.0934x over previous
import functools

import jax
import jax.numpy as jnp
from jax import lax
from jax.experimental import pallas as pl
from jax.experimental.pallas import tpu as pltpu
from jax.experimental.pallas import tpu_sc as plsc

N, E, D, H = 10000, 320000, 128, 64
NC, NS = 2, 16
NW = NC * NS
EPW = E // NW
CHUNK = 128
NFULL = EPW // CHUNK
REM = EPW - NFULL * CHUNK
N_PAD = 10240
RPT = N_PAD // NS


def _sc_agg_body(g_hbm, src_hbm, dst_hbm, zeros_hbm, out_hbm,
                 sidx, didx, rows, sidx2, didx2, rows2, obuf, table, sem):
    c = lax.axis_index("c")
    s = lax.axis_index("s")
    wid = s * NC + c
    base = wid * EPW
    r0 = s * RPT

    pltpu.sync_copy(zeros_hbm.at[pl.ds(r0, RPT)], table.at[pl.ds(r0, RPT)])
    plsc.subcore_barrier()

    def chunk_body(j, carry):
        off = pl.multiple_of(base + j * CHUNK, CHUNK)
        pltpu.sync_copy(src_hbm.at[pl.ds(off, CHUNK)], sidx)
        pltpu.sync_copy(dst_hbm.at[pl.ds(off, CHUNK)], didx)
        pltpu.async_copy(g_hbm.at[sidx], rows, sem).wait()
        pltpu.sync_copy(rows, table.at[didx], add=True)
        return carry

    lax.fori_loop(0, NFULL, chunk_body, 0)

    off = pl.multiple_of(base + NFULL * CHUNK, 8)
    pltpu.sync_copy(src_hbm.at[pl.ds(off, REM)], sidx2)
    pltpu.sync_copy(dst_hbm.at[pl.ds(off, REM)], didx2)
    pltpu.async_copy(g_hbm.at[sidx2], rows2, sem).wait()
    pltpu.sync_copy(rows2, table.at[didx2], add=True)

    plsc.subcore_barrier()
    pltpu.sync_copy(table.at[pl.ds(r0, RPT)], obuf)
    pltpu.sync_copy(obuf, out_hbm.at[c, pl.ds(r0, RPT)])


@jax.jit
def _sc_agg(g, src, dst, zeros):
    mesh = plsc.VectorSubcoreMesh(core_axis_name="c", subcore_axis_name="s")
    return pl.kernel(
        _sc_agg_body,
        out_type=jax.ShapeDtypeStruct((NC, N_PAD, H), jnp.float32),
        mesh=mesh,
        compiler_params=pltpu.CompilerParams(use_tc_tiling_on_sc=False),
        scratch_types=[
            pltpu.VMEM((CHUNK,), jnp.int32),
            pltpu.VMEM((CHUNK,), jnp.int32),
            pltpu.VMEM((CHUNK, H), jnp.float32),
            pltpu.VMEM((REM,), jnp.int32),
            pltpu.VMEM((REM,), jnp.int32),
            pltpu.VMEM((REM, H), jnp.float32),
            pltpu.VMEM((RPT, H), jnp.float32),
            pltpu.VMEM_SHARED((N_PAD, H), jnp.float32),
            pltpu.SemaphoreType.DMA,
        ],
    )(g, src, dst, zeros)


BR = 1000
GRID = N // BR


def _mm_body(x_ref, w_ref, o_ref):
    o_ref[...] = jnp.dot(x_ref[...], w_ref[...],
                         preferred_element_type=jnp.float32)


@jax.jit
def _tc_pre(x, W1):
    return pl.pallas_call(
        _mm_body,
        grid=(GRID,),
        in_specs=[
            pl.BlockSpec((BR, D), lambda i: (i, 0)),
            pl.BlockSpec((D, H), lambda i: (0, 0)),
        ],
        out_specs=pl.BlockSpec((BR, H), lambda i: (i, 0)),
        out_shape=jax.ShapeDtypeStruct((N, H), jnp.float32),
    )(x, W1)


def _mid_body(g_ref, p_ref, b1_ref, w2_ref, b2_ref, w1n_ref, o_ref):
    z = g_ref[...] + p_ref[0] + p_ref[1] + b1_ref[...]
    z = jnp.maximum(z, 0.0)
    h = jnp.dot(z, w2_ref[...], preferred_element_type=jnp.float32) + b2_ref[...]
    h = jnp.maximum(h, 0.0)
    o_ref[...] = jnp.dot(h, w1n_ref[...], preferred_element_type=jnp.float32)


@jax.jit
def _tc_mid(g, p, b1, W2, b2, W1n):
    return pl.pallas_call(
        _mid_body,
        grid=(GRID,),
        in_specs=[
            pl.BlockSpec((BR, H), lambda i: (i, 0)),
            pl.BlockSpec((NC, BR, H), lambda i: (0, i, 0)),
            pl.BlockSpec((1, H), lambda i: (0, 0)),
            pl.BlockSpec((H, H), lambda i: (0, 0)),
            pl.BlockSpec((1, H), lambda i: (0, 0)),
            pl.BlockSpec((H, H), lambda i: (0, 0)),
        ],
        out_specs=pl.BlockSpec((BR, H), lambda i: (i, 0)),
        out_shape=jax.ShapeDtypeStruct((N, H), jnp.float32),
    )(g, p, b1, W2, b2, W1n)


def _final_body(g_ref, p_ref, b1_ref, w2_ref, b2_ref, wf1_ref, bf1_ref,
                wf2_ref, bf2_ref, o_ref, acc_ref):
    i = pl.program_id(0)
    z = g_ref[...] + p_ref[0] + p_ref[1] + b1_ref[...]
    z = jnp.maximum(z, 0.0)
    h = jnp.dot(z, w2_ref[...], preferred_element_type=jnp.float32) + b2_ref[...]
    h = jnp.maximum(h, 0.0)
    part = jnp.sum(h, axis=0, keepdims=True)

    @pl.when(i == 0)
    def _():
        acc_ref[...] = jnp.zeros_like(acc_ref)

    acc_ref[0:1, :] += part

    @pl.when(i == GRID - 1)
    def _():
        pooled = acc_ref[0:1, :]
        t = jnp.maximum(
            jnp.dot(pooled, wf1_ref[...], preferred_element_type=jnp.float32)
            + bf1_ref[...], 0.0)
        o_ref[...] = (jnp.dot(t, wf2_ref[...],
                              preferred_element_type=jnp.float32)
                      + bf2_ref[...])


@jax.jit
def _tc_final(g, p, b1, W2, b2, Wf1, bf1, Wf2, bf2):
    return pl.pallas_call(
        _final_body,
        grid=(GRID,),
        in_specs=[
            pl.BlockSpec((BR, H), lambda i: (i, 0)),
            pl.BlockSpec((NC, BR, H), lambda i: (0, i, 0)),
            pl.BlockSpec((1, H), lambda i: (0, 0)),
            pl.BlockSpec((H, H), lambda i: (0, 0)),
            pl.BlockSpec((1, H), lambda i: (0, 0)),
            pl.BlockSpec((H, H), lambda i: (0, 0)),
            pl.BlockSpec((1, H), lambda i: (0, 0)),
            pl.BlockSpec((H, 1), lambda i: (0, 0)),
            pl.BlockSpec((1, 1), lambda i: (0, 0)),
        ],
        out_specs=pl.BlockSpec((1, 1), lambda i: (0, 0)),
        out_shape=jax.ShapeDtypeStruct((1, 1), jnp.float32),
        scratch_shapes=[pltpu.VMEM((8, H), jnp.float32)],
    )(g, p, b1, W2, b2, Wf1, bf1, Wf2, bf2)


def kernel(x, edge_index, W1_0, b1_0, W2_0, b2_0, W1_1, b1_1, W2_1, b2_1,
           W1_2, b1_2, W2_2, b2_2, Wf1, bf1, Wf2, bf2):
    src = edge_index[0]
    dst = edge_index[1]
    zeros = jnp.zeros((N_PAD, H), jnp.float32)

    g = _tc_pre(x, W1_0)
    p = _sc_agg(g, src, dst, zeros)
    g = _tc_mid(g, p, b1_0.reshape(1, H), W2_0, b2_0.reshape(1, H), W1_1)
    p = _sc_agg(g, src, dst, zeros)
    g = _tc_mid(g, p, b1_1.reshape(1, H), W2_1, b2_1.reshape(1, H), W1_2)
    p = _sc_agg(g, src, dst, zeros)
    out = _tc_final(g, p, b1_2.reshape(1, H), W2_2, b2_2.reshape(1, H),
                    Wf1, bf1.reshape(1, H), Wf2, bf2.reshape(1, 1))
    return out.reshape(-1)
